# baseline (speedup 1.0000x reference)
import jax
import jax.numpy as jnp
from jax import lax
from jax.experimental import pallas as pl
from jax.experimental.pallas import tpu as pltpu
from jax.experimental.pallas import tpu_sc as plsc

BATCH = 16384
HIST = 50
D = 64
NC, NS = 2, 16
NW = NC * NS
BPW = BATCH // NW
K = 4
NSS = BPW // K


def _body(lut_hbm, x_hbm, out_hbm, idx_a, idx_b, g_a, g_b,
          gsem_a, gsem_b, osem_a, osem_b, isem_a, isem_b):
    wid = lax.axis_index("s") * NC + lax.axis_index("c")
    base = wid * BPW
    idx = (idx_a, idx_b)
    g = (g_a, g_b)
    gsem = (gsem_a, gsem_b)
    osem = (osem_a, osem_b)
    isem = (isem_a, isem_b)

    def fire_idx(ss, p):
        for k in range(K):
            pltpu.async_copy(x_hbm.at[base + ss * K + k], idx[p].at[k],
                             isem[p])

    def drain_idx(p):
        for k in range(K):
            pltpu.make_async_copy(x_hbm.at[0], idx[p].at[k], isem[p]).wait()

    def fire_gather(ss, p):
        for k in range(K):
            pltpu.async_copy(lut_hbm.at[idx[p].at[k]], g[p].at[k], gsem[p])

    def drain_gather(p):
        for k in range(K):
            pltpu.make_async_copy(lut_hbm.at[idx[p].at[0]], g[p].at[k],
                                  gsem[p]).wait()

    def fire_write(ss, p):
        for k in range(K):
            pltpu.async_copy(g[p].at[k], out_hbm.at[base + ss * K + k],
                             osem[p])

    def drain_write(p):
        for k in range(K):
            pltpu.make_async_copy(g[p].at[k], out_hbm.at[0], osem[p]).wait()

    fire_idx(0, 0)
    drain_idx(0)
    fire_gather(0, 0)
    fire_idx(1, 1)
    drain_idx(1)
    drain_gather(0)
    fire_write(0, 0)
    fire_gather(1, 1)
    fire_idx(2, 0)

    @pl.loop(0, NSS - 2, step=2)
    def _(i):
        for q in (0, 1):
            ss = i + 1 + q
            p = (1 + q) % 2
            drain_idx(1 - p)
            drain_gather(p)
            fire_write(ss, p)
            drain_write(1 - p)
            fire_gather(ss + 1, 1 - p)

            @pl.when(ss + 2 < NSS)
            def _():
                fire_idx(ss + 2, p)

    drain_gather(1)
    fire_write(NSS - 1, 1)
    drain_write(0)
    drain_write(1)


@jax.jit
def _call(lut, x):
    mesh = plsc.VectorSubcoreMesh(core_axis_name="c", subcore_axis_name="s",
                                  num_cores=NC, num_subcores=NS)
    return pl.kernel(
        _body,
        out_type=jax.ShapeDtypeStruct((BATCH, HIST, D), jnp.float32),
        mesh=mesh,
        scratch_types=[
            pltpu.VMEM((K, HIST), jnp.int32),
            pltpu.VMEM((K, HIST), jnp.int32),
            pltpu.VMEM((K, HIST, D), jnp.float32),
            pltpu.VMEM((K, HIST, D), jnp.float32),
            pltpu.SemaphoreType.DMA,
            pltpu.SemaphoreType.DMA,
            pltpu.SemaphoreType.DMA,
            pltpu.SemaphoreType.DMA,
            pltpu.SemaphoreType.DMA,
            pltpu.SemaphoreType.DMA,
        ],
        compiler_params=pltpu.CompilerParams(use_tc_tiling_on_sc=False),
    )(lut, x)


def kernel(x, lut):
    return _call(lut, x)
